# baseline (speedup 1.0000x reference)
import jax
import jax.numpy as jnp
import numpy as np
from jax.experimental import pallas as pl
from jax.experimental.pallas import tpu as pltpu

_B = 32
_N = 256
_E = 1024
_D = 128
_H = 4
_DH = 32
_SLOTS = 80
_SPEC_C = 5
_SDIM = 64
_LP = 512
_LPF = 10
_LP_HID = 64
_LP_OUT = 32
_LATENT = 128

_CONV_TILE = 256


def _conv_edge_kernel(x_ref, lst_ref, w1_ref, b1_ref, w2_ref, b2_ref,
                      w3_ref, b3_ref, wes_ref, west_ref, be_ref, out_ref):
    T = _CONV_TILE
    x = x_ref[...]
    P = [x[:, j * 11:(j + 1) * 11, :] for j in range(8)]
    blk = lambda a: P[a % 8][:, (a // 8):(a // 8) + 10, :]
    c1 = jnp.concatenate(
        [jnp.concatenate([blk(2 * r + k) for k in range(5)], axis=2)
         for r in range(4)], axis=1)
    y1 = jnp.maximum(
        jnp.dot(c1.reshape(T * 40, 25), w1_ref[...],
                preferred_element_type=jnp.float32) + b1_ref[...], 0.0)
    y1 = y1.reshape(T, 40, 32)
    Y = [y1[:, r * 10:(r + 1) * 10, :] for r in range(4)]
    zf = jnp.zeros((T, 1, 32), jnp.float32)
    shm = lambda a: jnp.concatenate([zf, a[:, :9, :]], axis=1)
    shp = lambda a: jnp.concatenate([a[:, 1:, :], zf], axis=1)
    colsQ0 = jnp.concatenate([shm(Y[2]), shm(Y[3]), Y[0], Y[1], Y[2]], axis=2)
    colsQ1 = jnp.concatenate([Y[0], Y[1], Y[2], Y[3], shp(Y[0])], axis=2)
    c2 = jnp.concatenate([colsQ0, colsQ1], axis=1)
    y2 = jnp.maximum(
        jnp.dot(c2.reshape(T * 20, 160), w2_ref[...],
                preferred_element_type=jnp.float32) + b2_ref[...], 0.0)
    y2 = y2.reshape(T, 20, 64)
    Q0 = y2[:, :10, :]
    Q1 = y2[:, 10:, :]
    zf2 = jnp.zeros((T, 1, 64), jnp.float32)
    Q1m = jnp.concatenate([zf2, Q1[:, :9, :]], axis=1)
    c3 = jnp.concatenate([Q1m, Q0, Q1], axis=2)
    y3 = jnp.maximum(
        jnp.dot(c3.reshape(T * 10, 192), w3_ref[...],
                preferred_element_type=jnp.float32) + b3_ref[...], 0.0)
    link_spec = jnp.mean(y3.reshape(T, 10, 64), axis=1)
    ef = (jnp.dot(link_spec, wes_ref[...], preferred_element_type=jnp.float32)
          + jnp.dot(lst_ref[...], west_ref[...],
                    preferred_element_type=jnp.float32)
          + be_ref[...])
    out_ref[...] = jnp.maximum(ef, 0.0)


def _gat_kernel(nf_ref, ef_ref, src_c_ref, dst_c_ref, src_r_ref, dst_r_ref,
                wn_ref, bn_ref, wq_ref, wk_ref, wv1_ref, wv2_ref, wea_ref,
                wo_ref, bo_ref, lg_ref, lb_ref, out_ref):
    ef = ef_ref[0]
    src_c = src_c_ref[0]
    dst_c = dst_c_ref[0]
    src_r = src_r_ref[0]
    dst_r = dst_r_ref[0]

    iota_en = jax.lax.broadcasted_iota(jnp.int32, (_E, _N), 1)
    iota_ne = jax.lax.broadcasted_iota(jnp.int32, (_N, _E), 0)
    oh_d = (dst_c == iota_en).astype(jnp.float32)
    oh_s = (src_c == iota_en).astype(jnp.float32)
    ohT_d = (dst_r == iota_ne).astype(jnp.float32)
    ohT_sum = ohT_d + (src_r == iota_ne).astype(jnp.float32)

    hs_i = jax.lax.broadcasted_iota(jnp.int32, (_D, _H), 0) // _DH
    hs_j = jax.lax.broadcasted_iota(jnp.int32, (_D, _H), 1)
    hsum = (hs_i == hs_j).astype(jnp.float32)
    he_i = jax.lax.broadcasted_iota(jnp.int32, (_H, _D), 0)
    he_j = jax.lax.broadcasted_iota(jnp.int32, (_H, _D), 1) // _DH
    hexp = (he_i == he_j).astype(jnp.float32)

    h = jnp.maximum(
        jnp.dot(nf_ref[0], wn_ref[...], preferred_element_type=jnp.float32)
        + bn_ref[...], 0.0)

    scale = 1.0 / np.sqrt(_DH).astype(np.float32)
    for l in range(3):
        q = jnp.dot(h, wq_ref[l], preferred_element_type=jnp.float32)
        k = jnp.dot(h, wk_ref[l], preferred_element_type=jnp.float32)
        qd = jnp.dot(oh_d, q, preferred_element_type=jnp.float32)
        ks = jnp.dot(oh_s, k, preferred_element_type=jnp.float32)
        attn = (jnp.dot(qd * ks, hsum, preferred_element_type=jnp.float32)
                * scale
                + jnp.dot(ef, wea_ref[l], preferred_element_type=jnp.float32))
        mx = jnp.max(attn)
        ex = jnp.exp(attn - mx)
        sm = jnp.dot(ohT_d, ex, preferred_element_type=jnp.float32)
        den = jnp.dot(oh_d, sm, preferred_element_type=jnp.float32)
        w = ex / jnp.maximum(den, 1e-8)
        wfull = jnp.dot(w, hexp, preferred_element_type=jnp.float32)
        vn = jnp.dot(h, wv1_ref[l], preferred_element_type=jnp.float32)
        vs = (jnp.dot(oh_s, vn, preferred_element_type=jnp.float32)
              + jnp.dot(ef, wv2_ref[l], preferred_element_type=jnp.float32))
        msg = wfull * vs
        agg = jnp.dot(ohT_sum, msg, preferred_element_type=jnp.float32)
        out = (jnp.dot(agg, wo_ref[l], preferred_element_type=jnp.float32)
               + bo_ref[l])
        r = h + out
        m = jnp.mean(r, axis=-1, keepdims=True)
        v = jnp.mean((r - m) ** 2, axis=-1, keepdims=True)
        h = (r - m) / jnp.sqrt(v + 1e-5) * lg_ref[l] + lb_ref[l]

    out_ref[0, 0] = jnp.mean(h, axis=0)


def _head_kernel(pooled_ref, lp_ref, wl1_ref, bl1_ref, wl2_ref, bl2_ref,
                 wp1a_ref, wp1b_ref, bp1_ref, wp2_ref, bp2_ref,
                 g_ref, bb_ref, out_ref):
    y = jnp.maximum(
        jnp.dot(lp_ref[...], wl1_ref[...], preferred_element_type=jnp.float32)
        + bl1_ref[...], 0.0)
    y = (jnp.dot(y, wl2_ref[...], preferred_element_type=jnp.float32)
         + bl2_ref[...])
    lp_sum = jnp.mean(y.reshape(_B, _LP, _LP_OUT), axis=1)
    z = jnp.maximum(
        jnp.dot(pooled_ref[...], wp1a_ref[...],
                preferred_element_type=jnp.float32)
        + jnp.dot(lp_sum, wp1b_ref[...], preferred_element_type=jnp.float32)
        + bp1_ref[...], 0.0)
    z = jnp.dot(z, wp2_ref[...], preferred_element_type=jnp.float32) + bp2_ref[...]
    out_ref[...] = z * g_ref[...] + bb_ref[...]


def _full(shape):
    nd = len(shape)
    return pl.BlockSpec(shape, lambda *a: (0,) * nd)


def kernel(node_feat, spectral, link_static, edge_index, node_mask, link_mask,
           lp_features, lp_mask, params):
    p = params
    f32 = jnp.float32
    BL = _B * _E

    xpad = jnp.pad(spectral.reshape(BL, _SPEC_C, _SLOTS),
                   ((0, 0), (0, 0), (2, 6))).reshape(BL, _SPEC_C * 88)
    j_i = np.arange(88)
    q = 8 * (j_i % 11) + (j_i // 11)
    idx = (np.arange(_SPEC_C)[None, :] * 88 + q[:, None]).astype(np.int32)
    xph = xpad[:, idx]
    lst = link_static.reshape(BL, 4)
    w1c = p['cw1'].transpose(2, 1, 0).reshape(25, 32)
    w2c = p['cw2'].transpose(2, 1, 0).reshape(160, 64)
    w3c = p['cw3'].transpose(2, 1, 0).reshape(192, 64)
    wes = p['We'][:, :_SDIM].T
    west = p['We'][:, _SDIM:].T
    nblk = BL // _CONV_TILE
    ef = pl.pallas_call(
        _conv_edge_kernel,
        grid=(nblk,),
        in_specs=[
            pl.BlockSpec((_CONV_TILE, 88, _SPEC_C), lambda i: (i, 0, 0)),
            pl.BlockSpec((_CONV_TILE, 4), lambda i: (i, 0)),
            _full((25, 32)), _full((1, 32)),
            _full((160, 64)), _full((1, 64)),
            _full((192, 64)), _full((1, 64)),
            _full((64, 128)), _full((4, 128)), _full((1, 128)),
        ],
        out_specs=pl.BlockSpec((_CONV_TILE, _D), lambda i: (i, 0)),
        out_shape=jax.ShapeDtypeStruct((BL, _D), f32),
    )(xph, lst, w1c, p['cb1'].reshape(1, 32), w2c, p['cb2'].reshape(1, 64),
      w3c, p['cb3'].reshape(1, 64), wes, west, p['be'].reshape(1, 128))

    ef = ef.reshape(_B, _E, _D)

    ei = edge_index.astype(jnp.int32)
    src_c = ei[:, :, 0:1]
    dst_c = ei[:, :, 1:2]
    src_r = ei[:, :, 0].reshape(_B, 1, _E)
    dst_r = ei[:, :, 1].reshape(_B, 1, _E)
    g = p['gat']
    wq = jnp.stack([g[l]['Wq'].T for l in range(3)])
    wk = jnp.stack([g[l]['Wk'].T for l in range(3)])
    wv1 = jnp.stack([g[l]['Wv'][:, :_D].T for l in range(3)])
    wv2 = jnp.stack([g[l]['Wv'][:, _D:].T for l in range(3)])
    wea = jnp.stack([g[l]['Wea'].T for l in range(3)])
    wo = jnp.stack([g[l]['Wo'].T for l in range(3)])
    bo = jnp.stack([g[l]['bo'].reshape(1, _D) for l in range(3)])
    lg = jnp.stack([g[l]['lg'].reshape(1, _D) for l in range(3)])
    lb = jnp.stack([g[l]['lb'].reshape(1, _D) for l in range(3)])

    pooled = pl.pallas_call(
        _gat_kernel,
        grid=(_B,),
        in_specs=[
            pl.BlockSpec((1, _N, 8), lambda i: (i, 0, 0)),
            pl.BlockSpec((1, _E, _D), lambda i: (i, 0, 0)),
            pl.BlockSpec((1, _E, 1), lambda i: (i, 0, 0)),
            pl.BlockSpec((1, _E, 1), lambda i: (i, 0, 0)),
            pl.BlockSpec((1, 1, _E), lambda i: (i, 0, 0)),
            pl.BlockSpec((1, 1, _E), lambda i: (i, 0, 0)),
            _full((8, _D)), _full((1, _D)),
            _full((3, _D, _D)), _full((3, _D, _D)),
            _full((3, _D, _D)), _full((3, _D, _D)),
            _full((3, _D, _H)),
            _full((3, _D, _D)), _full((3, 1, _D)),
            _full((3, 1, _D)), _full((3, 1, _D)),
        ],
        out_specs=pl.BlockSpec((1, 1, _D), lambda i: (i, 0, 0)),
        out_shape=jax.ShapeDtypeStruct((_B, 1, _D), f32),
    )(node_feat, ef, src_c, dst_c, src_r, dst_r,
      p['Wn'].T, p['bn'].reshape(1, _D), wq, wk, wv1, wv2, wea, wo, bo, lg, lb)

    lp2 = lp_features.reshape(_B * _LP, _LPF)
    gscale = (p['bng'] / np.sqrt(1.0 + 1e-5)).reshape(1, _LATENT)
    z = pl.pallas_call(
        _head_kernel,
        in_specs=[
            _full((_B, _D)), _full((_B * _LP, _LPF)),
            _full((_LPF, _LP_HID)), _full((1, _LP_HID)),
            _full((_LP_HID, _LP_OUT)), _full((1, _LP_OUT)),
            _full((_D, _LATENT)), _full((_LP_OUT, _LATENT)),
            _full((1, _LATENT)),
            _full((_LATENT, _LATENT)), _full((1, _LATENT)),
            _full((1, _LATENT)), _full((1, _LATENT)),
        ],
        out_specs=_full((_B, _LATENT)),
        out_shape=jax.ShapeDtypeStruct((_B, _LATENT), f32),
    )(pooled.reshape(_B, _D), lp2, p['Wl1'].T, p['bl1'].reshape(1, _LP_HID),
      p['Wl2'].T, p['bl2'].reshape(1, _LP_OUT),
      p['Wp1'][:, :_D].T, p['Wp1'][:, _D:].T, p['bp1'].reshape(1, _LATENT),
      p['Wp2'].T, p['bp2'].reshape(1, _LATENT),
      gscale, p['bnb'].reshape(1, _LATENT))
    return z


# speedup vs baseline: 1.7252x; 1.7252x over previous
import jax
import jax.numpy as jnp
import numpy as np
from jax.experimental import pallas as pl
from jax.experimental.pallas import tpu as pltpu

_B = 32
_N = 256
_E = 1024
_D = 128
_H = 4
_DH = 32
_SLOTS = 80
_SPEC_C = 5
_SDIM = 64
_LP = 512
_LPF = 10
_LP_HID = 64
_LP_OUT = 32
_LATENT = 128

_CONV_TILE = 512


def _conv_edge_kernel(x_ref, lst_ref, w1_ref, b1_ref, w2_ref, b2_ref,
                      w3_ref, b3_ref, wes_ref, west_ref, be_ref, out_ref):
    T = _CONV_TILE
    c1 = x_ref[...]
    y1 = jnp.maximum(
        jnp.dot(c1.reshape(T * 40, 25), w1_ref[...],
                preferred_element_type=jnp.float32) + b1_ref[...], 0.0)
    y1 = y1.reshape(T, 40, 32)
    Y = [y1[:, r * 10:(r + 1) * 10, :] for r in range(4)]
    zf = jnp.zeros((T, 1, 32), jnp.float32)
    shm = lambda a: jnp.concatenate([zf, a[:, :9, :]], axis=1)
    shp = lambda a: jnp.concatenate([a[:, 1:, :], zf], axis=1)
    colsQ0 = jnp.concatenate([shm(Y[2]), shm(Y[3]), Y[0], Y[1], Y[2]], axis=2)
    colsQ1 = jnp.concatenate([Y[0], Y[1], Y[2], Y[3], shp(Y[0])], axis=2)
    c2 = jnp.concatenate([colsQ0, colsQ1], axis=1)
    y2 = jnp.maximum(
        jnp.dot(c2.reshape(T * 20, 160), w2_ref[...],
                preferred_element_type=jnp.float32) + b2_ref[...], 0.0)
    y2 = y2.reshape(T, 20, 64)
    Q0 = y2[:, :10, :]
    Q1 = y2[:, 10:, :]
    zf2 = jnp.zeros((T, 1, 64), jnp.float32)
    Q1m = jnp.concatenate([zf2, Q1[:, :9, :]], axis=1)
    c3 = jnp.concatenate([Q1m, Q0, Q1], axis=2)
    y3 = jnp.maximum(
        jnp.dot(c3.reshape(T * 10, 192), w3_ref[...],
                preferred_element_type=jnp.float32) + b3_ref[...], 0.0)
    link_spec = jnp.mean(y3.reshape(T, 10, 64), axis=1)
    ef = (jnp.dot(link_spec, wes_ref[...], preferred_element_type=jnp.float32)
          + jnp.dot(lst_ref[...], west_ref[...],
                    preferred_element_type=jnp.float32)
          + be_ref[...])
    out_ref[...] = jnp.maximum(ef, 0.0)


def _gat_kernel(nf_ref, ef_ref, src_c_ref, dst_c_ref, src_r_ref, dst_r_ref,
                wn_ref, bn_ref, wq_ref, wk_ref, wv1_ref, wv2_ref, wea_ref,
                wo_ref, bo_ref, lg_ref, lb_ref, out_ref):
    ef = ef_ref[0]
    src_c = src_c_ref[0]
    dst_c = dst_c_ref[0]
    src_r = src_r_ref[0]
    dst_r = dst_r_ref[0]

    iota_en = jax.lax.broadcasted_iota(jnp.int32, (_E, _N), 1)
    iota_ne = jax.lax.broadcasted_iota(jnp.int32, (_N, _E), 0)
    oh_d = (dst_c == iota_en).astype(jnp.float32)
    oh_s = (src_c == iota_en).astype(jnp.float32)
    ohT_d = (dst_r == iota_ne).astype(jnp.float32)
    ohT_sum = ohT_d + (src_r == iota_ne).astype(jnp.float32)

    hs_i = jax.lax.broadcasted_iota(jnp.int32, (_D, _H), 0) // _DH
    hs_j = jax.lax.broadcasted_iota(jnp.int32, (_D, _H), 1)
    hsum = (hs_i == hs_j).astype(jnp.float32)
    he_i = jax.lax.broadcasted_iota(jnp.int32, (_H, _D), 0)
    he_j = jax.lax.broadcasted_iota(jnp.int32, (_H, _D), 1) // _DH
    hexp = (he_i == he_j).astype(jnp.float32)

    h = jnp.maximum(
        jnp.dot(nf_ref[0], wn_ref[...], preferred_element_type=jnp.float32)
        + bn_ref[...], 0.0)

    scale = 1.0 / np.sqrt(_DH).astype(np.float32)
    for l in range(3):
        q = jnp.dot(h, wq_ref[l], preferred_element_type=jnp.float32)
        k = jnp.dot(h, wk_ref[l], preferred_element_type=jnp.float32)
        qd = jnp.dot(oh_d, q, preferred_element_type=jnp.float32)
        ks = jnp.dot(oh_s, k, preferred_element_type=jnp.float32)
        attn = (jnp.dot(qd * ks, hsum, preferred_element_type=jnp.float32)
                * scale
                + jnp.dot(ef, wea_ref[l], preferred_element_type=jnp.float32))
        mx = jnp.max(attn)
        ex = jnp.exp(attn - mx)
        sm = jnp.dot(ohT_d, ex, preferred_element_type=jnp.float32)
        den = jnp.dot(oh_d, sm, preferred_element_type=jnp.float32)
        w = ex / jnp.maximum(den, 1e-8)
        wfull = jnp.dot(w, hexp, preferred_element_type=jnp.float32)
        vn = jnp.dot(h, wv1_ref[l], preferred_element_type=jnp.float32)
        vs = (jnp.dot(oh_s, vn, preferred_element_type=jnp.float32)
              + jnp.dot(ef, wv2_ref[l], preferred_element_type=jnp.float32))
        msg = wfull * vs
        agg = jnp.dot(ohT_sum, msg, preferred_element_type=jnp.float32)
        out = (jnp.dot(agg, wo_ref[l], preferred_element_type=jnp.float32)
               + bo_ref[l])
        r = h + out
        m = jnp.mean(r, axis=-1, keepdims=True)
        v = jnp.mean((r - m) ** 2, axis=-1, keepdims=True)
        h = (r - m) / jnp.sqrt(v + 1e-5) * lg_ref[l] + lb_ref[l]

    out_ref[0, 0] = jnp.mean(h, axis=0)


def _head_kernel(pooled_ref, lp_ref, wl1_ref, bl1_ref, wl2_ref, bl2_ref,
                 wp1a_ref, wp1b_ref, bp1_ref, wp2_ref, bp2_ref,
                 g_ref, bb_ref, out_ref):
    y = jnp.maximum(
        jnp.dot(lp_ref[...], wl1_ref[...], preferred_element_type=jnp.float32)
        + bl1_ref[...], 0.0)
    y = (jnp.dot(y, wl2_ref[...], preferred_element_type=jnp.float32)
         + bl2_ref[...])
    lp_sum = jnp.mean(y.reshape(_B, _LP, _LP_OUT), axis=1)
    z = jnp.maximum(
        jnp.dot(pooled_ref[...], wp1a_ref[...],
                preferred_element_type=jnp.float32)
        + jnp.dot(lp_sum, wp1b_ref[...], preferred_element_type=jnp.float32)
        + bp1_ref[...], 0.0)
    z = jnp.dot(z, wp2_ref[...], preferred_element_type=jnp.float32) + bp2_ref[...]
    out_ref[...] = z * g_ref[...] + bb_ref[...]


def _full(shape):
    nd = len(shape)
    return pl.BlockSpec(shape, lambda *a: (0,) * nd)


def kernel(node_feat, spectral, link_static, edge_index, node_mask, link_mask,
           lp_features, lp_mask, params):
    p = params
    f32 = jnp.float32
    BL = _B * _E

    xz = jnp.concatenate(
        [spectral.reshape(BL, _SPEC_C * _SLOTS),
         jnp.zeros((BL, 8), f32)], axis=1)
    t_idx = (np.arange(4)[:, None] + 4 * np.arange(10)[None, :]).reshape(40)
    pos = 2 * t_idx[:, None] - 2 + np.arange(5)[None, :]
    cidx = np.arange(_SPEC_C)
    idx = cidx[None, None, :] * _SLOTS + pos[:, :, None]
    valid = ((pos >= 0) & (pos < _SLOTS))[:, :, None]
    idx = np.where(valid, idx, _SPEC_C * _SLOTS)
    idx = idx.reshape(40, 25).astype(np.int32)
    cols1 = xz[:, idx]
    lst = link_static.reshape(BL, 4)
    w1c = p['cw1'].transpose(2, 1, 0).reshape(25, 32)
    w2c = p['cw2'].transpose(2, 1, 0).reshape(160, 64)
    w3c = p['cw3'].transpose(2, 1, 0).reshape(192, 64)
    wes = p['We'][:, :_SDIM].T
    west = p['We'][:, _SDIM:].T
    nblk = BL // _CONV_TILE
    ef = pl.pallas_call(
        _conv_edge_kernel,
        grid=(nblk,),
        in_specs=[
            pl.BlockSpec((_CONV_TILE, 40, 25), lambda i: (i, 0, 0)),
            pl.BlockSpec((_CONV_TILE, 4), lambda i: (i, 0)),
            _full((25, 32)), _full((1, 32)),
            _full((160, 64)), _full((1, 64)),
            _full((192, 64)), _full((1, 64)),
            _full((64, 128)), _full((4, 128)), _full((1, 128)),
        ],
        out_specs=pl.BlockSpec((_CONV_TILE, _D), lambda i: (i, 0)),
        out_shape=jax.ShapeDtypeStruct((BL, _D), f32),
    )(cols1, lst, w1c, p['cb1'].reshape(1, 32), w2c, p['cb2'].reshape(1, 64),
      w3c, p['cb3'].reshape(1, 64), wes, west, p['be'].reshape(1, 128))

    ef = ef.reshape(_B, _E, _D)

    ei = edge_index.astype(jnp.int32)
    src_c = ei[:, :, 0:1]
    dst_c = ei[:, :, 1:2]
    src_r = ei[:, :, 0].reshape(_B, 1, _E)
    dst_r = ei[:, :, 1].reshape(_B, 1, _E)
    g = p['gat']
    wq = jnp.stack([g[l]['Wq'].T for l in range(3)])
    wk = jnp.stack([g[l]['Wk'].T for l in range(3)])
    wv1 = jnp.stack([g[l]['Wv'][:, :_D].T for l in range(3)])
    wv2 = jnp.stack([g[l]['Wv'][:, _D:].T for l in range(3)])
    wea = jnp.stack([g[l]['Wea'].T for l in range(3)])
    wo = jnp.stack([g[l]['Wo'].T for l in range(3)])
    bo = jnp.stack([g[l]['bo'].reshape(1, _D) for l in range(3)])
    lg = jnp.stack([g[l]['lg'].reshape(1, _D) for l in range(3)])
    lb = jnp.stack([g[l]['lb'].reshape(1, _D) for l in range(3)])

    pooled = pl.pallas_call(
        _gat_kernel,
        grid=(_B,),
        in_specs=[
            pl.BlockSpec((1, _N, 8), lambda i: (i, 0, 0)),
            pl.BlockSpec((1, _E, _D), lambda i: (i, 0, 0)),
            pl.BlockSpec((1, _E, 1), lambda i: (i, 0, 0)),
            pl.BlockSpec((1, _E, 1), lambda i: (i, 0, 0)),
            pl.BlockSpec((1, 1, _E), lambda i: (i, 0, 0)),
            pl.BlockSpec((1, 1, _E), lambda i: (i, 0, 0)),
            _full((8, _D)), _full((1, _D)),
            _full((3, _D, _D)), _full((3, _D, _D)),
            _full((3, _D, _D)), _full((3, _D, _D)),
            _full((3, _D, _H)),
            _full((3, _D, _D)), _full((3, 1, _D)),
            _full((3, 1, _D)), _full((3, 1, _D)),
        ],
        out_specs=pl.BlockSpec((1, 1, _D), lambda i: (i, 0, 0)),
        out_shape=jax.ShapeDtypeStruct((_B, 1, _D), f32),
    )(node_feat, ef, src_c, dst_c, src_r, dst_r,
      p['Wn'].T, p['bn'].reshape(1, _D), wq, wk, wv1, wv2, wea, wo, bo, lg, lb)

    lp2 = lp_features.reshape(_B * _LP, _LPF)
    gscale = (p['bng'] / np.sqrt(1.0 + 1e-5)).reshape(1, _LATENT)
    z = pl.pallas_call(
        _head_kernel,
        in_specs=[
            _full((_B, _D)), _full((_B * _LP, _LPF)),
            _full((_LPF, _LP_HID)), _full((1, _LP_HID)),
            _full((_LP_HID, _LP_OUT)), _full((1, _LP_OUT)),
            _full((_D, _LATENT)), _full((_LP_OUT, _LATENT)),
            _full((1, _LATENT)),
            _full((_LATENT, _LATENT)), _full((1, _LATENT)),
            _full((1, _LATENT)), _full((1, _LATENT)),
        ],
        out_specs=_full((_B, _LATENT)),
        out_shape=jax.ShapeDtypeStruct((_B, _LATENT), f32),
    )(pooled.reshape(_B, _D), lp2, p['Wl1'].T, p['bl1'].reshape(1, _LP_HID),
      p['Wl2'].T, p['bl2'].reshape(1, _LP_OUT),
      p['Wp1'][:, :_D].T, p['Wp1'][:, _D:].T, p['bp1'].reshape(1, _LATENT),
      p['Wp2'].T, p['bp2'].reshape(1, _LATENT),
      gscale, p['bnb'].reshape(1, _LATENT))
    return z
